# baseline (speedup 1.0000x reference)
import functools

import jax
import jax.numpy as jnp
from jax import lax
from jax.experimental import pallas as pl
from jax.experimental.pallas import tpu as pltpu
from jax.experimental.pallas import tpu_sc as plsc

N = 50000
E = 800000
NP = 50176
NTRASH = 64
CH = 128

EPS = 50176
NCH_M = EPS // CH
NPAIR2 = NCH_M // 4

NPH = NP // 2
DTRASH = 256
DACC = NPH + DTRASH
DPS = DACC // 16

ROWS_PER_SUB = NP // 16
OCH = 32
NOCH = ROWS_PER_SUB // OCH

BR = 3136
GRID = NP // BR


_MESH = plsc.VectorSubcoreMesh(core_axis_name="c", subcore_axis_name="s")
_SC_PARAMS = pltpu.CompilerParams(use_tc_tiling_on_sc=False)


def _deg_body(idx_hbm, degp_hbm, idxb_v, ones_v, zbuf_v, acc_sh,
              semi0, semi1, sems):
    c = lax.axis_index("c")
    s = lax.axis_index("s")
    half0 = c * NPH

    def _z(i, _):
        zbuf_v[pl.ds(i * 16, 16)] = jnp.zeros((16,), jnp.float32)
        return _

    lax.fori_loop(0, DPS // 16, _z, None)
    for i in range(CH // 16):
        ones_v[pl.ds(i * 16, 16)] = jnp.full((16,), 1.0, jnp.float32)
    pltpu.sync_copy(zbuf_v, acc_sh.at[pl.ds(s * DPS, DPS)])
    plsc.subcore_barrier()

    def _ldq(q, sl, sm):
        pltpu.async_copy(idx_hbm.at[s, pl.ds(4 * q, 4)], idxb_v.at[sl], sm)

    def _wtq(q, sl, sm):
        pltpu.make_async_copy(idx_hbm.at[s, pl.ds(4 * q, 4)],
                              idxb_v.at[sl], sm).wait()

    def _remap(sl):
        for u in range(4):
            for t in range(CH // 16):
                d = idxb_v[sl, u, 1, pl.ds(t * 16, 16)]
                inhalf = (d >= half0) & (d < half0 + NPH)
                idxb_v[sl, u, 1, pl.ds(t * 16, 16)] = jnp.where(
                    inhalf, d - half0, NPH + (d & (DTRASH - 1)))

    def _fire(sl):
        for u in range(4):
            pltpu.async_copy(ones_v, acc_sh.at[idxb_v.at[sl, u, 1]], sems,
                             add=True)

    def _drain(sl):
        for u in range(4):
            pltpu.make_async_copy(ones_v, acc_sh.at[idxb_v.at[sl, u, 1]],
                                  sems).wait()

    NQ = NCH_M // 4
    _ldq(0, 0, semi0)
    _ldq(1, 1, semi1)

    def _body(i, _):
        a = 2 * i
        _wtq(a, 0, semi0)
        _remap(0)
        _fire(0)
        _drain(0)

        @pl.when(a + 2 < NQ)
        def _():
            _ldq(a + 2, 0, semi0)

        _wtq(a + 1, 1, semi1)
        _remap(1)
        _fire(1)
        _drain(1)

        @pl.when(a + 3 < NQ)
        def _():
            _ldq(a + 3, 1, semi1)

        return _

    lax.fori_loop(0, NQ // 2, _body, None)
    plsc.subcore_barrier()
    pltpu.sync_copy(acc_sh.at[pl.ds(s * DPS, DPS)], zbuf_v)
    pltpu.sync_copy(zbuf_v, degp_hbm.at[pl.ds(c * DACC + s * DPS, DPS)])


_deg_call = functools.partial(
    pl.kernel,
    out_type=jax.ShapeDtypeStruct((2 * DACC,), jnp.float32),
    mesh=_MESH,
    compiler_params=_SC_PARAMS,
    scratch_types=[
        pltpu.VMEM((2, 4, 2, CH), jnp.int32),
        pltpu.VMEM((CH,), jnp.float32),
        pltpu.VMEM((DPS,), jnp.float32),
        pltpu.VMEM_SHARED((DACC,), jnp.float32),
        pltpu.SemaphoreType.DMA,
        pltpu.SemaphoreType.DMA,
        pltpu.SemaphoreType.DMA,
    ],
)(_deg_body)


def _scat_body(g_hbm, idx_hbm, zer_hbm, out_hbm,
               idxb_v, rows_v, stage_v, zbuf_v, oidx_v, acc_sh,
               semi0, semi1, semg0, semg1, semg2, semg3):
    c = lax.axis_index("c")
    s = lax.axis_index("s")
    iota2 = lax.iota(jnp.int32, 16) * 2
    base2 = (s * ROWS_PER_SUB) * 2 + c

    def _oidx(k, _):
        for t in range(OCH // 16):
            oidx_v[k, pl.ds(t * 16, 16)] = (
                base2 + k * (OCH * 2) + t * 32 + iota2)
        return _

    lax.fori_loop(0, NOCH, _oidx, None)
    pltpu.sync_copy(zer_hbm, zbuf_v)

    def _zero(k, _):
        pltpu.sync_copy(zbuf_v, acc_sh.at[pl.ds(s * ROWS_PER_SUB + k * OCH,
                                                OCH)])
        return _

    lax.fori_loop(0, NOCH, _zero, None)
    plsc.subcore_barrier()

    semg = (semg0, semg1, semg2, semg3)
    g_view = g_hbm.at[pl.ds(c, 2 * NP - 8)]

    def _ldq(q, sl, sm):
        pltpu.async_copy(idx_hbm.at[s, pl.ds(4 * q, 4)], idxb_v.at[sl], sm)

    def _wtq(q, sl, sm):
        pltpu.make_async_copy(idx_hbm.at[s, pl.ds(4 * q, 4)],
                              idxb_v.at[sl], sm).wait()

    def _g(sl, u):
        pltpu.async_copy(g_view.at[idxb_v.at[sl, u, 0]], rows_v.at[u],
                         semg[u])

    def _wg(sl, u):
        pltpu.make_async_copy(g_view.at[idxb_v.at[sl, u, 0]], rows_v.at[u],
                              semg[u]).wait()

    def _sc(sl, u):
        pltpu.sync_copy(rows_v.at[u], acc_sh.at[idxb_v.at[sl, u, 1]],
                        add=True)

    NQ = NCH_M // 4
    _ldq(0, 0, semi0)
    _wtq(0, 0, semi0)
    for u in range(4):
        _g(0, u)
    _ldq(1, 1, semi1)

    def _body(i, _):
        q0 = 2 * i
        _wtq(q0 + 1, 1, semi1)
        for u in range(4):
            _wg(0, u)
            _sc(0, u)
            _g(1, u)

        @pl.when(q0 + 2 < NQ)
        def _():
            _ldq(q0 + 2, 0, semi0)

        for u in range(4):
            _wg(1, u)
            _sc(1, u)

        @pl.when(q0 + 2 < NQ)
        def _():
            _wtq(q0 + 2, 0, semi0)
            for u in range(4):
                _g(0, u)

        @pl.when(q0 + 3 < NQ)
        def _():
            _ldq(q0 + 3, 1, semi1)

        return _

    lax.fori_loop(0, NQ // 2, _body, None)
    plsc.subcore_barrier()

    def _out(k, _):
        pltpu.sync_copy(acc_sh.at[pl.ds(s * ROWS_PER_SUB + k * OCH, OCH)],
                        stage_v)
        pltpu.sync_copy(stage_v, out_hbm.at[oidx_v.at[k]])
        return _

    lax.fori_loop(0, NOCH, _out, None)


_scat_call = functools.partial(
    pl.kernel,
    out_type=jax.ShapeDtypeStruct((2 * NP, 32), jnp.float32),
    mesh=_MESH,
    compiler_params=_SC_PARAMS,
    scratch_types=[
        pltpu.VMEM((2, 4, 2, CH), jnp.int32),
        pltpu.VMEM((4, CH, 32), jnp.float32),
        pltpu.VMEM((OCH, 32), jnp.float32),
        pltpu.VMEM((OCH, 32), jnp.float32),
        pltpu.VMEM((NOCH, OCH), jnp.int32),
        pltpu.VMEM_SHARED((NP, 32), jnp.float32),
        pltpu.SemaphoreType.DMA,
        pltpu.SemaphoreType.DMA,
        pltpu.SemaphoreType.DMA,
        pltpu.SemaphoreType.DMA,
        pltpu.SemaphoreType.DMA,
        pltpu.SemaphoreType.DMA,
    ],
)(_scat_body)


def _dinv(deg):
    return lax.rsqrt(deg + 1.0)


def _tc1_body(x_ref, wct_ref, bc_ref, w1_ref, degp_ref, out_ref):
    din = _dinv(degp_ref[...])
    t = jnp.dot(x_ref[...], wct_ref[...], preferred_element_type=jnp.float32)
    t = t + bc_ref[...][None, :]
    out_ref[...] = jnp.dot(t, w1_ref[...],
                           preferred_element_type=jnp.float32) * din


def _tc2_body(s_ref, g_ref, degp_ref, b1_ref, w2_ref, out_ref):
    din = _dinv(degp_ref[...])
    h = (s_ref[...] + g_ref[...]) * din + b1_ref[...][None, :]
    h = jnp.maximum(h, 0.0)
    out_ref[...] = jnp.dot(h, w2_ref[...],
                           preferred_element_type=jnp.float32) * din


def _tc3_body(s_ref, g_ref, degp_ref, b2_ref, out_ref):
    din = _dinv(degp_ref[...])
    out_ref[...] = (s_ref[...] + g_ref[...]) * din + b2_ref[...][None, :]


def _row_spec(w):
    return pl.BlockSpec((BR, w), lambda i: (i, 0))


def _degp_spec():
    return pl.BlockSpec((BR, 1), lambda i: (i, 0))


def _full_spec(shape):
    nd = len(shape)
    return pl.BlockSpec(shape, lambda i: (0,) * nd)


_tc1 = pl.pallas_call(
    _tc1_body,
    out_shape=jax.ShapeDtypeStruct((NP, 64), jnp.float32),
    grid=(GRID,),
    in_specs=[_row_spec(24), _full_spec((24, 64)), _full_spec((64,)),
              _full_spec((64, 64)), _degp_spec()],
    out_specs=_row_spec(64),
)

_tc2 = pl.pallas_call(
    _tc2_body,
    out_shape=jax.ShapeDtypeStruct((NP, 64), jnp.float32),
    grid=(GRID,),
    in_specs=[_row_spec(64), _row_spec(64), _degp_spec(),
              _full_spec((64,)), _full_spec((64, 64))],
    out_specs=_row_spec(64),
)

_tc3 = pl.pallas_call(
    _tc3_body,
    out_shape=jax.ShapeDtypeStruct((NP, 64), jnp.float32),
    grid=(GRID,),
    in_specs=[_row_spec(64), _row_spec(64), _degp_spec(), _full_spec((64,))],
    out_specs=_row_spec(64),
)


def kernel(x, edge_index, conv1d_w, conv1d_b, W1, b1, W2, b2):
    xf = x[:, :, 0]
    xp = jnp.zeros((NP, 24), jnp.float32).at[:N].set(xf)
    wct = conv1d_w[:, 0, :].T
    zer = jnp.zeros((OCH, 32), jnp.float32)

    src = edge_index[0]
    dst = edge_index[1]

    npad = EPS - E // 16
    fill_s = (jnp.arange(16 * npad, dtype=jnp.int32).reshape(16, -1)
              * 9973) % N
    srcm = jnp.concatenate([src.reshape(16, E // 16), fill_s], axis=1)
    srcm = srcm.reshape(16, NCH_M, 1, CH)
    fill_t = N + (jnp.arange(16 * npad, dtype=jnp.int32)
                  .reshape(16, -1) % NTRASH)
    dstm = jnp.concatenate([dst.reshape(16, E // 16), fill_t], axis=1)
    dstm = dstm.reshape(16, NCH_M, 1, CH)
    idx_all = jnp.concatenate([srcm * 2, dstm], axis=2)

    degp = _deg_call(idx_all)
    degp3 = degp.reshape(2, DACC)[:, :NPH].reshape(NP, 1)

    g1 = _tc1(xp, wct, conv1d_b, W1, degp3)
    s1 = _scat_call(g1.reshape(2 * NP, 32), idx_all, zer)
    g2 = _tc2(s1.reshape(NP, 64), g1, degp3, b1, W2)
    s2 = _scat_call(g2.reshape(2 * NP, 32), idx_all, zer)
    out = _tc3(s2.reshape(NP, 64), g2, degp3, b2)
    return out[:N]


# speedup vs baseline: 1.0259x; 1.0259x over previous
import functools

import jax
import jax.numpy as jnp
from jax import lax
from jax.experimental import pallas as pl
from jax.experimental.pallas import tpu as pltpu
from jax.experimental.pallas import tpu_sc as plsc

N = 50000
E = 800000
NP = 50176
NTRASH = 64
CH = 128

EPS = 50176
NCH_M = EPS // CH
NPAIR2 = NCH_M // 4

NPH = NP // 2
DTRASH = 256
DACC = NPH + DTRASH
DPS = DACC // 16

ROWS_PER_SUB = NP // 16
OCH = 32
NOCH = ROWS_PER_SUB // OCH

BR = 3136
GRID = NP // BR


_MESH = plsc.VectorSubcoreMesh(core_axis_name="c", subcore_axis_name="s")
_SC_PARAMS = pltpu.CompilerParams(use_tc_tiling_on_sc=False)


def _deg_body(idx_hbm, degp_hbm, idxb_v, ones_v, zbuf_v, acc_sh,
              semi0, semi1, sems):
    c = lax.axis_index("c")
    s = lax.axis_index("s")
    half0 = c * NPH

    def _z(i, _):
        zbuf_v[pl.ds(i * 16, 16)] = jnp.zeros((16,), jnp.float32)
        return _

    lax.fori_loop(0, DPS // 16, _z, None)
    for i in range(CH // 16):
        ones_v[pl.ds(i * 16, 16)] = jnp.full((16,), 1.0, jnp.float32)
    pltpu.sync_copy(zbuf_v, acc_sh.at[pl.ds(s * DPS, DPS)])
    plsc.subcore_barrier()

    def _ldq(q, sl, sm):
        pltpu.async_copy(idx_hbm.at[s, pl.ds(4 * q, 4)], idxb_v.at[sl], sm)

    def _wtq(q, sl, sm):
        pltpu.make_async_copy(idx_hbm.at[s, pl.ds(4 * q, 4)],
                              idxb_v.at[sl], sm).wait()

    def _remap(sl):
        for u in range(4):
            for t in range(CH // 16):
                d = idxb_v[sl, u, 1, pl.ds(t * 16, 16)]
                inhalf = (d >= half0) & (d < half0 + NPH)
                idxb_v[sl, u, 1, pl.ds(t * 16, 16)] = jnp.where(
                    inhalf, d - half0, NPH + (d & (DTRASH - 1)))

    def _fire(sl):
        for u in range(4):
            pltpu.async_copy(ones_v, acc_sh.at[idxb_v.at[sl, u, 1]], sems,
                             add=True)

    def _drain(sl):
        for u in range(4):
            pltpu.make_async_copy(ones_v, acc_sh.at[idxb_v.at[sl, u, 1]],
                                  sems).wait()

    NQ = NCH_M // 4
    _ldq(0, 0, semi0)
    _ldq(1, 1, semi1)

    def _body(i, _):
        a = 2 * i
        _wtq(a, 0, semi0)
        _remap(0)
        _fire(0)
        _drain(0)

        @pl.when(a + 2 < NQ)
        def _():
            _ldq(a + 2, 0, semi0)

        _wtq(a + 1, 1, semi1)
        _remap(1)
        _fire(1)
        _drain(1)

        @pl.when(a + 3 < NQ)
        def _():
            _ldq(a + 3, 1, semi1)

        return _

    lax.fori_loop(0, NQ // 2, _body, None)
    plsc.subcore_barrier()
    pltpu.sync_copy(acc_sh.at[pl.ds(s * DPS, DPS)], zbuf_v)
    pltpu.sync_copy(zbuf_v, degp_hbm.at[pl.ds(c * DACC + s * DPS, DPS)])


_deg_call = functools.partial(
    pl.kernel,
    out_type=jax.ShapeDtypeStruct((2 * DACC,), jnp.float32),
    mesh=_MESH,
    compiler_params=_SC_PARAMS,
    scratch_types=[
        pltpu.VMEM((2, 4, 2, CH), jnp.int32),
        pltpu.VMEM((CH,), jnp.float32),
        pltpu.VMEM((DPS,), jnp.float32),
        pltpu.VMEM_SHARED((DACC,), jnp.float32),
        pltpu.SemaphoreType.DMA,
        pltpu.SemaphoreType.DMA,
        pltpu.SemaphoreType.DMA,
    ],
)(_deg_body)


def _scat_body(g_hbm, idx_hbm, zer_hbm, out_hbm,
               idxb_v, rows_v, stage_v, oidx_v, acc_sh,
               semi0, semi1, semg0, semg1, semg2, semg3):
    c = lax.axis_index("c")
    s = lax.axis_index("s")
    iota2 = lax.iota(jnp.int32, 16) * 2
    base2 = (s * ROWS_PER_SUB) * 2 + c

    def _oidx(k, _):
        for t in range(OCH // 16):
            oidx_v[k, pl.ds(t * 16, 16)] = (
                base2 + k * (OCH * 2) + t * 32 + iota2)
        return _

    lax.fori_loop(0, NOCH, _oidx, None)
    pltpu.sync_copy(zer_hbm.at[pl.ds(s * ROWS_PER_SUB, ROWS_PER_SUB)],
                    acc_sh.at[pl.ds(s * ROWS_PER_SUB, ROWS_PER_SUB)])
    plsc.subcore_barrier()

    semg = (semg0, semg1, semg2, semg3)
    g_view = g_hbm.at[pl.ds(c, 2 * NP - 8)]

    def _ldq(q, sl, sm):
        pltpu.async_copy(idx_hbm.at[s, pl.ds(4 * q, 4)], idxb_v.at[sl], sm)

    def _wtq(q, sl, sm):
        pltpu.make_async_copy(idx_hbm.at[s, pl.ds(4 * q, 4)],
                              idxb_v.at[sl], sm).wait()

    def _g(sl, u):
        pltpu.async_copy(g_view.at[idxb_v.at[sl, u, 0]], rows_v.at[u],
                         semg[u])

    def _wg(sl, u):
        pltpu.make_async_copy(g_view.at[idxb_v.at[sl, u, 0]], rows_v.at[u],
                              semg[u]).wait()

    def _sc(sl, u):
        pltpu.sync_copy(rows_v.at[u], acc_sh.at[idxb_v.at[sl, u, 1]],
                        add=True)

    NQ = NCH_M // 4
    _ldq(0, 0, semi0)
    _wtq(0, 0, semi0)
    for u in range(4):
        _g(0, u)
    _ldq(1, 1, semi1)

    def _body(i, _):
        q0 = 2 * i
        _wtq(q0 + 1, 1, semi1)
        for u in range(4):
            _wg(0, u)
            _sc(0, u)
            _g(1, u)

        @pl.when(q0 + 2 < NQ)
        def _():
            _ldq(q0 + 2, 0, semi0)

        for u in range(4):
            _wg(1, u)
            _sc(1, u)

        @pl.when(q0 + 2 < NQ)
        def _():
            _wtq(q0 + 2, 0, semi0)
            for u in range(4):
                _g(0, u)

        @pl.when(q0 + 3 < NQ)
        def _():
            _ldq(q0 + 3, 1, semi1)

        return _

    lax.fori_loop(0, NQ // 2, _body, None)
    plsc.subcore_barrier()

    def _ldst(k, b, sm):
        pltpu.async_copy(acc_sh.at[pl.ds(s * ROWS_PER_SUB + k * OCH, OCH)],
                         stage_v.at[b], sm)

    def _wlst(k, b, sm):
        pltpu.make_async_copy(
            acc_sh.at[pl.ds(s * ROWS_PER_SUB + k * OCH, OCH)],
            stage_v.at[b], sm).wait()

    def _fout(k, b, sm):
        pltpu.async_copy(stage_v.at[b], out_hbm.at[oidx_v.at[k]], sm)

    def _wout(k, b, sm):
        pltpu.make_async_copy(stage_v.at[b], out_hbm.at[oidx_v.at[k]],
                              sm).wait()

    _ldst(0, 0, semg0)
    _ldst(1, 1, semg1)

    def _out(i, _):
        k0 = 2 * i
        _wlst(k0, 0, semg0)
        _fout(k0, 0, semg2)
        _wlst(k0 + 1, 1, semg1)
        _fout(k0 + 1, 1, semg3)

        @pl.when(k0 + 2 < NOCH)
        def _():
            _wout(k0, 0, semg2)
            _ldst(k0 + 2, 0, semg0)

        @pl.when(k0 + 3 < NOCH)
        def _():
            _wout(k0 + 1, 1, semg3)
            _ldst(k0 + 3, 1, semg1)

        return _

    lax.fori_loop(0, NOCH // 2, _out, None)
    _wout(NOCH - 2, 0, semg2)
    _wout(NOCH - 1, 1, semg3)


_scat_call = functools.partial(
    pl.kernel,
    out_type=jax.ShapeDtypeStruct((2 * NP, 32), jnp.float32),
    mesh=_MESH,
    compiler_params=_SC_PARAMS,
    scratch_types=[
        pltpu.VMEM((2, 4, 2, CH), jnp.int32),
        pltpu.VMEM((4, CH, 32), jnp.float32),
        pltpu.VMEM((2, OCH, 32), jnp.float32),
        pltpu.VMEM((NOCH, OCH), jnp.int32),
        pltpu.VMEM_SHARED((NP, 32), jnp.float32),
        pltpu.SemaphoreType.DMA,
        pltpu.SemaphoreType.DMA,
        pltpu.SemaphoreType.DMA,
        pltpu.SemaphoreType.DMA,
        pltpu.SemaphoreType.DMA,
        pltpu.SemaphoreType.DMA,
    ],
)(_scat_body)


def _dinv(deg):
    return lax.rsqrt(deg + 1.0)


def _tc1_body(x_ref, wct_ref, bc_ref, w1_ref, degp_ref, out_ref):
    din = _dinv(degp_ref[...])
    t = jnp.dot(x_ref[...], wct_ref[...], preferred_element_type=jnp.float32)
    t = t + bc_ref[...][None, :]
    out_ref[...] = jnp.dot(t, w1_ref[...],
                           preferred_element_type=jnp.float32) * din


def _tc2_body(s_ref, g_ref, degp_ref, b1_ref, w2_ref, out_ref):
    din = _dinv(degp_ref[...])
    h = (s_ref[...] + g_ref[...]) * din + b1_ref[...][None, :]
    h = jnp.maximum(h, 0.0)
    out_ref[...] = jnp.dot(h, w2_ref[...],
                           preferred_element_type=jnp.float32) * din


def _tc3_body(s_ref, g_ref, degp_ref, b2_ref, out_ref):
    din = _dinv(degp_ref[...])
    out_ref[...] = (s_ref[...] + g_ref[...]) * din + b2_ref[...][None, :]


def _row_spec(w):
    return pl.BlockSpec((BR, w), lambda i: (i, 0))


def _degp_spec():
    return pl.BlockSpec((BR, 1), lambda i: (i, 0))


def _full_spec(shape):
    nd = len(shape)
    return pl.BlockSpec(shape, lambda i: (0,) * nd)


_tc1 = pl.pallas_call(
    _tc1_body,
    out_shape=jax.ShapeDtypeStruct((NP, 64), jnp.float32),
    grid=(GRID,),
    in_specs=[_row_spec(24), _full_spec((24, 64)), _full_spec((64,)),
              _full_spec((64, 64)), _degp_spec()],
    out_specs=_row_spec(64),
)

_tc2 = pl.pallas_call(
    _tc2_body,
    out_shape=jax.ShapeDtypeStruct((NP, 64), jnp.float32),
    grid=(GRID,),
    in_specs=[_row_spec(64), _row_spec(64), _degp_spec(),
              _full_spec((64,)), _full_spec((64, 64))],
    out_specs=_row_spec(64),
)

_tc3 = pl.pallas_call(
    _tc3_body,
    out_shape=jax.ShapeDtypeStruct((NP, 64), jnp.float32),
    grid=(GRID,),
    in_specs=[_row_spec(64), _row_spec(64), _degp_spec(), _full_spec((64,))],
    out_specs=_row_spec(64),
)


def kernel(x, edge_index, conv1d_w, conv1d_b, W1, b1, W2, b2):
    xf = x[:, :, 0]
    xp = jnp.zeros((NP, 24), jnp.float32).at[:N].set(xf)
    wct = conv1d_w[:, 0, :].T
    zer = jnp.zeros((NP, 32), jnp.float32)

    src = edge_index[0]
    dst = edge_index[1]

    npad = EPS - E // 16
    fill_s = (jnp.arange(16 * npad, dtype=jnp.int32).reshape(16, -1)
              * 9973) % N
    srcm = jnp.concatenate([src.reshape(16, E // 16), fill_s], axis=1)
    srcm = srcm.reshape(16, NCH_M, 1, CH)
    fill_t = N + (jnp.arange(16 * npad, dtype=jnp.int32)
                  .reshape(16, -1) % NTRASH)
    dstm = jnp.concatenate([dst.reshape(16, E // 16), fill_t], axis=1)
    dstm = dstm.reshape(16, NCH_M, 1, CH)
    idx_all = jnp.concatenate([srcm * 2, dstm], axis=2)

    degp = _deg_call(idx_all)
    degp3 = degp.reshape(2, DACC)[:, :NPH].reshape(NP, 1)

    g1 = _tc1(xp, wct, conv1d_b, W1, degp3)
    s1 = _scat_call(g1.reshape(2 * NP, 32), idx_all, zer)
    g2 = _tc2(s1.reshape(NP, 64), g1, degp3, b1, W2)
    s2 = _scat_call(g2.reshape(2 * NP, 32), idx_all, zer)
    out = _tc3(s2.reshape(NP, 64), g2, degp3, b2)
    return out[:N]
